# Initial kernel scaffold; baseline (speedup 1.0000x reference)
import jax
import jax.numpy as jnp
from jax.experimental import pallas as pl


def kernel(x_bill, x_bill_version, x_legislator_term, x_legislator, x_committee, x_party, x_topic, ts_bill, ts_bill_version, ts_legislator_term, ei_has_version, ei_voted_on, ei_serves, ei_about, ea_voted_on, t2v_w0, t2v_w, t2v_b, prj_ln_g_bill, prj_ln_b_bill, prj_W_bill, prj_ln_g_bill_version, prj_ln_b_bill_version, prj_W_bill_version, prj_ln_g_legislator_term, prj_ln_b_legislator_term, prj_W_legislator_term, prj_ln_g_legislator, prj_ln_b_legislator, prj_W_legislator, prj_ln_g_committee, prj_ln_b_committee, prj_W_committee, prj_ln_g_party, prj_ln_b_party, prj_W_party, prj_ln_g_topic, prj_ln_b_topic, prj_W_topic, Q_0, K_0, V_0, rel_0_has_version, rel_0_voted_on, rel_0_serves, rel_0_about, ffn_ln_g_0, ffn_ln_b_0, ffn_W1_0, ffn_W2_0, Q_1, K_1, V_1, rel_1_has_version, rel_1_voted_on, rel_1_serves, rel_1_about, ffn_ln_g_1, ffn_ln_b_1, ffn_W1_1, ffn_W2_1, Q_2, K_2, V_2, rel_2_has_version, rel_2_voted_on, rel_2_serves, rel_2_about, ffn_ln_g_2, ffn_ln_b_2, ffn_W1_2, ffn_W2_2, vote_W1, vote_b1, vote_W2, vote_b2, norm_g_bill, norm_b_bill, norm_g_bill_version, norm_b_bill_version, norm_g_legislator_term, norm_b_legislator_term, norm_g_legislator, norm_b_legislator, norm_g_committee, norm_b_committee, norm_g_party, norm_b_party, norm_g_topic, norm_b_topic):
    raise NotImplementedError("write your pallas kernel here")


# speedup vs baseline: 1.5430x; 1.5430x over previous
import functools

import numpy as np
import jax
import jax.numpy as jnp
from jax import lax
from jax.experimental import pallas as pl
from jax.experimental.pallas import tpu as pltpu
from jax.experimental.pallas import tpu_sc as plsc

D = 192
H = 4
DK = 48
_ISQ = float(1.0 / np.sqrt(DK))
_NODE_TYPES = ["bill", "bill_version", "legislator_term", "legislator",
               "committee", "party", "topic"]
_IN_DIMS = {"bill": 389, "bill_version": 769, "legislator_term": 385,
            "legislator": 2, "committee": 385, "party": 384, "topic": 384}
_TIME_TYPES = ("bill", "bill_version", "legislator_term")
_EDGE_TYPES = [("bill", "has_version", "bill_version"),
               ("legislator_term", "voted_on", "bill_version"),
               ("legislator", "serves", "legislator_term"),
               ("bill", "about", "topic")]

_SEL = np.zeros((D, 8), np.float32)
for _d in range(D):
    _SEL[_d, _d // DK] = 1.0
_SELT = _SEL.T.copy()


def _ru(x, m):
    return (x + m - 1) // m * m


def _pad2(a, r, c):
    return jnp.pad(a, ((0, r - a.shape[0]), (0, c - a.shape[1])))


def _ln_exact(x, g, b):
    mu = jnp.mean(x, axis=1, keepdims=True)
    xc = x - mu
    var = jnp.mean(xc * xc, axis=1, keepdims=True)
    return xc * lax.rsqrt(var + 1e-5) * g + b


def _gelu(x):
    return 0.5 * x * (1.0 + lax.erf(x * np.float32(1.0 / np.sqrt(2.0))))


def _proj_body(timed, d_x, x_ref, wf_ref, bf_ref, g_ref, b_ref, gt_ref,
               bt_ref, Wx_ref, Wt_ref, o_ref):
    xb = x_ref[...]
    colx = lax.broadcasted_iota(jnp.int32, xb.shape, 1)
    xm = jnp.where(colx < d_x, xb, 0.0)
    s1 = jnp.sum(xm, axis=1, keepdims=True)
    s2 = jnp.sum(xm * xm, axis=1, keepdims=True)
    if timed:
        t = xb[:, d_x:d_x + 1]
        tv = t * wf_ref[...] + bf_ref[...]
        c16 = lax.broadcasted_iota(jnp.int32, tv.shape, 1)
        v = jnp.where(c16 == 0, tv, jnp.sin(tv))
        v = jnp.where(c16 < 12, v, 0.0)
        s1 = s1 + jnp.sum(v, axis=1, keepdims=True)
        s2 = s2 + jnp.sum(v * v, axis=1, keepdims=True)
        n = d_x + 12
    else:
        n = d_x
    mu = s1 / n
    var = jnp.maximum(s2 / n - mu * mu, 0.0)
    r = lax.rsqrt(var + 1e-5)
    zx = ((xm - mu) * r) * g_ref[...] + b_ref[...]
    zx = jnp.where(colx < d_x, zx, 0.0)
    acc = jnp.dot(zx, Wx_ref[...], preferred_element_type=jnp.float32)
    if timed:
        zv = ((v - mu) * r) * gt_ref[...] + bt_ref[...]
        acc = acc + jnp.dot(zv, Wt_ref[...], preferred_element_type=jnp.float32)
    o_ref[...] = _gelu(acc)


def _project(x, ts, g, b, W, w0, wt, bt, timed):
    N, d_x = x.shape
    B = 256
    P = _ru(d_x + 1, 128)
    Np = _ru(max(N, 8), B)
    if timed:
        xc = jnp.concatenate([x, ts[:, None]], axis=1)
        wf = jnp.concatenate([w0[None], wt, jnp.zeros((4,), jnp.float32)])[None]
        bf = jnp.concatenate([bt, jnp.zeros((4,), jnp.float32)])[None]
        gt = jnp.pad(g[d_x:d_x + 12], (0, 4))[None]
        btv = jnp.pad(b[d_x:d_x + 12], (0, 4))[None]
        Wt = jnp.pad(W[d_x:d_x + 12], ((0, 4), (0, 0)))
    else:
        xc = x
        wf = jnp.zeros((1, 16), jnp.float32)
        bf = jnp.zeros((1, 16), jnp.float32)
        gt = jnp.zeros((1, 16), jnp.float32)
        btv = jnp.zeros((1, 16), jnp.float32)
        Wt = jnp.zeros((16, D), jnp.float32)
    xp = _pad2(xc, Np, P)
    gp = jnp.pad(g[:d_x], (0, P - d_x))[None]
    bp = jnp.pad(b[:d_x], (0, P - d_x))[None]
    Wx = _pad2(W[:d_x], P, D)
    full = lambda i: (0, 0)
    out = pl.pallas_call(
        functools.partial(_proj_body, timed, d_x),
        grid=(Np // B,),
        in_specs=[pl.BlockSpec((B, P), lambda i: (i, 0)),
                  pl.BlockSpec((1, 16), full), pl.BlockSpec((1, 16), full),
                  pl.BlockSpec((1, P), full), pl.BlockSpec((1, P), full),
                  pl.BlockSpec((1, 16), full), pl.BlockSpec((1, 16), full),
                  pl.BlockSpec((P, D), full), pl.BlockSpec((16, D), full)],
        out_specs=pl.BlockSpec((B, D), lambda i: (i, 0)),
        out_shape=jax.ShapeDtypeStruct((Np, D), jnp.float32),
    )(xp, wf, bf, gp, bp, gt, btv, Wx, Wt)
    return out[:N]


def _mm_body(x_ref, w_ref, o_ref):
    o_ref[...] = jnp.dot(x_ref[...], w_ref[...],
                         preferred_element_type=jnp.float32)


def _matmul(x, W):
    N, K = x.shape
    M = W.shape[1]
    B = 512
    Np = _ru(max(N, 8), B)
    xp = _pad2(x, Np, K)
    out = pl.pallas_call(
        _mm_body,
        grid=(Np // B,),
        in_specs=[pl.BlockSpec((B, K), lambda i: (i, 0)),
                  pl.BlockSpec((K, M), lambda i: (0, 0))],
        out_specs=pl.BlockSpec((B, M), lambda i: (i, 0)),
        out_shape=jax.ShapeDtypeStruct((Np, M), jnp.float32),
    )(xp, W)
    return out[:N]


def _edge_body(q_ref, kv_ref, r_ref, sel_ref, selT_ref, ev_ref, el_ref):
    q = q_ref[...]
    k = kv_ref[:, :D] + r_ref[...]
    v = kv_ref[:, D:]
    lg = jnp.dot(q * k, sel_ref[...], preferred_element_type=jnp.float32) * _ISQ
    lg = jnp.where(lg > 0, lg, 0.01 * lg)
    col = lax.broadcasted_iota(jnp.int32, lg.shape, 1)
    el = jnp.where(col < H, jnp.exp(lg), 0.0)
    el_ref[...] = el
    ev_ref[...] = v * jnp.dot(el, selT_ref[...],
                              preferred_element_type=jnp.float32)


def _edge_vals(qe, kve, rflat):
    E = qe.shape[0]
    B = 512
    Ep = _ru(max(E, 8), B)
    qp = _pad2(qe, Ep, D)
    kvp = _pad2(kve, Ep, 2 * D)
    full = lambda i: (0, 0)
    ev, el = pl.pallas_call(
        _edge_body,
        grid=(Ep // B,),
        in_specs=[pl.BlockSpec((B, D), lambda i: (i, 0)),
                  pl.BlockSpec((B, 2 * D), lambda i: (i, 0)),
                  pl.BlockSpec((1, D), full),
                  pl.BlockSpec((D, 8), full), pl.BlockSpec((8, D), full)],
        out_specs=[pl.BlockSpec((B, D), lambda i: (i, 0)),
                   pl.BlockSpec((B, 8), lambda i: (i, 0))],
        out_shape=[jax.ShapeDtypeStruct((Ep, D), jnp.float32),
                   jax.ShapeDtypeStruct((Ep, 8), jnp.float32)],
    )(qp, kvp, rflat, jnp.asarray(_SEL), jnp.asarray(_SELT))
    return ev[:E], el[:E]


def _node_body(n_msgs, h_ref, *refs):
    rest = list(refs)
    P1 = rest[:2 * n_msgs]
    selT_ref, g_ref, b_ref, W1_ref, W2_ref, o_ref = rest[2 * n_msgs:]
    h = h_ref[...]
    msg = jnp.zeros_like(h)
    for i in range(n_msgs):
        Pm = P1[2 * i][...]
        sm = P1[2 * i + 1][...]
        den = jnp.dot(sm, selT_ref[...],
                      preferred_element_type=jnp.float32) + 1e-16
        msg = msg + Pm / den
    hr = h + msg
    one = jnp.ones((1, D), jnp.float32)
    zero = jnp.zeros((1, D), jnp.float32)
    z = _ln_exact(hr, one, zero)
    z = _ln_exact(z, g_ref[...], b_ref[...])
    ff = jnp.dot(_gelu(jnp.dot(z, W1_ref[...],
                               preferred_element_type=jnp.float32)),
                 W2_ref[...], preferred_element_type=jnp.float32)
    o_ref[...] = hr + ff


def _node_update(h, msgs, g, b, W1, W2):
    N = h.shape[0]
    B = 256
    Np = _ru(max(N, 8), B)
    hp = _pad2(h, Np, D)
    args = [hp]
    in_specs = [pl.BlockSpec((B, D), lambda i: (i, 0))]
    for (Pm, sm) in msgs:
        args.append(_pad2(Pm, Np, D))
        in_specs.append(pl.BlockSpec((B, D), lambda i: (i, 0)))
        args.append(_pad2(sm, Np, 8))
        in_specs.append(pl.BlockSpec((B, 8), lambda i: (i, 0)))
    full = lambda i: (0, 0)
    args += [jnp.asarray(_SELT), g[None], b[None], W1, W2]
    in_specs += [pl.BlockSpec((8, D), full), pl.BlockSpec((1, D), full),
                 pl.BlockSpec((1, D), full), pl.BlockSpec((D, 4 * D), full),
                 pl.BlockSpec((4 * D, D), full)]
    out = pl.pallas_call(
        functools.partial(_node_body, len(msgs)),
        grid=(Np // B,),
        in_specs=in_specs,
        out_specs=pl.BlockSpec((B, D), lambda i: (i, 0)),
        out_shape=jax.ShapeDtypeStruct((Np, D), jnp.float32),
    )(*args)
    return out[:N]


def _vote_body(raw_ref, pol_ref, hg_ref, W1_ref, b1_ref, W2_ref, b2_ref,
               o_ref):
    e1 = jnp.maximum(jnp.dot(raw_ref[...], W1_ref[...],
                             preferred_element_type=jnp.float32) + b1_ref[...],
                     0.0)
    pol = jnp.clip(pol_ref[...], 0.0, 1.0)
    ef = (jnp.dot(e1, W2_ref[...], preferred_element_type=jnp.float32)
          + b2_ref[...]) * (pol + 0.01)
    o_ref[...] = ef * hg_ref[...]


def _vote_vals(raw, pol, hg, W1, b1, W2, b2):
    E = raw.shape[0]
    B = 512
    Ep = _ru(E, B)
    full = lambda i: (0, 0)
    out = pl.pallas_call(
        _vote_body,
        grid=(Ep // B,),
        in_specs=[pl.BlockSpec((B, 384), lambda i: (i, 0)),
                  pl.BlockSpec((B, 1), lambda i: (i, 0)),
                  pl.BlockSpec((B, D), lambda i: (i, 0)),
                  pl.BlockSpec((384, D), full), pl.BlockSpec((1, D), full),
                  pl.BlockSpec((D, D), full), pl.BlockSpec((1, D), full)],
        out_specs=pl.BlockSpec((B, D), lambda i: (i, 0)),
        out_shape=jax.ShapeDtypeStruct((Ep, D), jnp.float32),
    )(_pad2(raw, Ep, 384), _pad2(pol, Ep, 1), _pad2(hg, Ep, D),
      W1, b1[None], W2, b2[None])
    return out[:E]


def _final_body(h_ref, g_ref, b_ref, o_ref):
    o_ref[...] = jnp.maximum(_ln_exact(h_ref[...], g_ref[...], b_ref[...]),
                             0.0)


def _final_norm(h, g, b):
    N = h.shape[0]
    B = 256
    Np = _ru(max(N, 8), B)
    full = lambda i: (0, 0)
    out = pl.pallas_call(
        _final_body,
        grid=(Np // B,),
        in_specs=[pl.BlockSpec((B, D), lambda i: (i, 0)),
                  pl.BlockSpec((1, D), full), pl.BlockSpec((1, D), full)],
        out_specs=pl.BlockSpec((B, D), lambda i: (i, 0)),
        out_shape=jax.ShapeDtypeStruct((Np, D), jnp.float32),
    )(_pad2(h, Np, D), g[None], b[None])
    return out[:N]


def _gather_rows(table, idx):
    return table[idx]


def _scatter_rows(vals, idx, n):
    return jax.ops.segment_sum(vals, idx, num_segments=n)


def kernel(x_bill, x_bill_version, x_legislator_term, x_legislator,
           x_committee, x_party, x_topic, ts_bill, ts_bill_version,
           ts_legislator_term, ei_has_version, ei_voted_on, ei_serves,
           ei_about, ea_voted_on, t2v_w0, t2v_w, t2v_b,
           prj_ln_g_bill, prj_ln_b_bill, prj_W_bill,
           prj_ln_g_bill_version, prj_ln_b_bill_version, prj_W_bill_version,
           prj_ln_g_legislator_term, prj_ln_b_legislator_term,
           prj_W_legislator_term, prj_ln_g_legislator, prj_ln_b_legislator,
           prj_W_legislator, prj_ln_g_committee, prj_ln_b_committee,
           prj_W_committee, prj_ln_g_party, prj_ln_b_party, prj_W_party,
           prj_ln_g_topic, prj_ln_b_topic, prj_W_topic,
           Q_0, K_0, V_0, rel_0_has_version, rel_0_voted_on, rel_0_serves,
           rel_0_about, ffn_ln_g_0, ffn_ln_b_0, ffn_W1_0, ffn_W2_0,
           Q_1, K_1, V_1, rel_1_has_version, rel_1_voted_on, rel_1_serves,
           rel_1_about, ffn_ln_g_1, ffn_ln_b_1, ffn_W1_1, ffn_W2_1,
           Q_2, K_2, V_2, rel_2_has_version, rel_2_voted_on, rel_2_serves,
           rel_2_about, ffn_ln_g_2, ffn_ln_b_2, ffn_W1_2, ffn_W2_2,
           vote_W1, vote_b1, vote_W2, vote_b2,
           norm_g_bill, norm_b_bill, norm_g_bill_version, norm_b_bill_version,
           norm_g_legislator_term, norm_b_legislator_term, norm_g_legislator,
           norm_b_legislator, norm_g_committee, norm_b_committee,
           norm_g_party, norm_b_party, norm_g_topic, norm_b_topic):
    p = dict(locals())
    h = {}
    for nt in _NODE_TYPES:
        timed = nt in _TIME_TYPES
        h[nt] = _project(p["x_" + nt],
                         p.get("ts_" + nt) if timed else None,
                         p["prj_ln_g_" + nt], p["prj_ln_b_" + nt],
                         p["prj_W_" + nt], t2v_w0, t2v_w, t2v_b, timed)
    nnodes = {nt: h[nt].shape[0] for nt in _NODE_TYPES}
    ei = {r: p["ei_" + r].astype(jnp.int32) for (_, r, _) in _EDGE_TYPES}
    src_types = {s for (s, _, _) in _EDGE_TYPES}
    dst_types = {t for (_, _, t) in _EDGE_TYPES}
    for l in range(3):
        Qw, Kw, Vw = p["Q_%d" % l], p["K_%d" % l], p["V_%d" % l]
        KVw = jnp.concatenate([Kw, Vw], axis=1)
        Qh = {t: _matmul(h[t], Qw) for t in dst_types}
        KVh = {s: _matmul(h[s], KVw) for s in src_types}
        msgs = {t: [] for t in _NODE_TYPES}
        for (sname, rname, tname) in _EDGE_TYPES:
            e = ei[rname]
            qe = _gather_rows(Qh[tname], e[1])
            kve = _gather_rows(KVh[sname], e[0])
            rflat = p["rel_%d_%s" % (l, rname)].reshape(1, D)
            ev, el = _edge_vals(qe, kve, rflat)
            Pn = _scatter_rows(ev, e[1], nnodes[tname])
            sn = _scatter_rows(el, e[1], nnodes[tname])
            msgs[tname].append((Pn, sn))
        h = {nt: _node_update(h[nt], msgs[nt], p["ffn_ln_g_%d" % l],
                              p["ffn_ln_b_%d" % l], p["ffn_W1_%d" % l],
                              p["ffn_W2_%d" % l])
             for nt in _NODE_TYPES}
    ev_ei = ei["voted_on"]
    hg = _gather_rows(h["legislator_term"], ev_ei[0])
    m = _vote_vals(ea_voted_on[:, 1:], ea_voted_on[:, :1], hg,
                   vote_W1, vote_b1, vote_W2, vote_b2)
    vmsg = _scatter_rows(m, ev_ei[1], nnodes["bill_version"])
    h["bill_version"] = h["bill_version"] + vmsg
    return tuple(_final_norm(h[nt], p["norm_g_" + nt], p["norm_b_" + nt])
                 for nt in _NODE_TYPES)


# speedup vs baseline: 1.8338x; 1.1885x over previous
import functools

import numpy as np
import jax
import jax.numpy as jnp
from jax import lax
from jax.experimental import pallas as pl
from jax.experimental.pallas import tpu as pltpu
from jax.experimental.pallas import tpu_sc as plsc

D = 192
H = 4
DK = 48
_ISQ = float(1.0 / np.sqrt(DK))
_NODE_TYPES = ["bill", "bill_version", "legislator_term", "legislator",
               "committee", "party", "topic"]
_IN_DIMS = {"bill": 389, "bill_version": 769, "legislator_term": 385,
            "legislator": 2, "committee": 385, "party": 384, "topic": 384}
_TIME_TYPES = ("bill", "bill_version", "legislator_term")
_EDGE_TYPES = [("bill", "has_version", "bill_version"),
               ("legislator_term", "voted_on", "bill_version"),
               ("legislator", "serves", "legislator_term"),
               ("bill", "about", "topic")]

_SEL = np.zeros((D, 8), np.float32)
for _d in range(D):
    _SEL[_d, _d // DK] = 1.0
_SELT = _SEL.T.copy()


def _ru(x, m):
    return (x + m - 1) // m * m


def _pad2(a, r, c):
    return jnp.pad(a, ((0, r - a.shape[0]), (0, c - a.shape[1])))


def _ln_exact(x, g, b):
    mu = jnp.mean(x, axis=1, keepdims=True)
    xc = x - mu
    var = jnp.mean(xc * xc, axis=1, keepdims=True)
    return xc * lax.rsqrt(var + 1e-5) * g + b


def _gelu(x):
    return 0.5 * x * (1.0 + lax.erf(x * np.float32(1.0 / np.sqrt(2.0))))


def _proj_body(timed, d_x, x_ref, wf_ref, bf_ref, g_ref, b_ref, gt_ref,
               bt_ref, Wx_ref, Wt_ref, o_ref):
    xb = x_ref[...]
    colx = lax.broadcasted_iota(jnp.int32, xb.shape, 1)
    xm = jnp.where(colx < d_x, xb, 0.0)
    s1 = jnp.sum(xm, axis=1, keepdims=True)
    s2 = jnp.sum(xm * xm, axis=1, keepdims=True)
    if timed:
        t = xb[:, d_x:d_x + 1]
        tv = t * wf_ref[...] + bf_ref[...]
        c16 = lax.broadcasted_iota(jnp.int32, tv.shape, 1)
        v = jnp.where(c16 == 0, tv, jnp.sin(tv))
        v = jnp.where(c16 < 12, v, 0.0)
        s1 = s1 + jnp.sum(v, axis=1, keepdims=True)
        s2 = s2 + jnp.sum(v * v, axis=1, keepdims=True)
        n = d_x + 12
    else:
        n = d_x
    mu = s1 / n
    var = jnp.maximum(s2 / n - mu * mu, 0.0)
    r = lax.rsqrt(var + 1e-5)
    zx = ((xm - mu) * r) * g_ref[...] + b_ref[...]
    zx = jnp.where(colx < d_x, zx, 0.0)
    acc = jnp.dot(zx, Wx_ref[...], preferred_element_type=jnp.float32)
    if timed:
        zv = ((v - mu) * r) * gt_ref[...] + bt_ref[...]
        acc = acc + jnp.dot(zv, Wt_ref[...], preferred_element_type=jnp.float32)
    o_ref[...] = _gelu(acc)


def _project(x, ts, g, b, W, w0, wt, bt, timed):
    N, d_x = x.shape
    B = 256
    P = _ru(d_x + 1, 128)
    Np = _ru(max(N, 8), B)
    if timed:
        xc = jnp.concatenate([x, ts[:, None]], axis=1)
        wf = jnp.concatenate([w0[None], wt, jnp.zeros((4,), jnp.float32)])[None]
        bf = jnp.concatenate([bt, jnp.zeros((4,), jnp.float32)])[None]
        gt = jnp.pad(g[d_x:d_x + 12], (0, 4))[None]
        btv = jnp.pad(b[d_x:d_x + 12], (0, 4))[None]
        Wt = jnp.pad(W[d_x:d_x + 12], ((0, 4), (0, 0)))
    else:
        xc = x
        wf = jnp.zeros((1, 16), jnp.float32)
        bf = jnp.zeros((1, 16), jnp.float32)
        gt = jnp.zeros((1, 16), jnp.float32)
        btv = jnp.zeros((1, 16), jnp.float32)
        Wt = jnp.zeros((16, D), jnp.float32)
    xp = _pad2(xc, Np, P)
    gp = jnp.pad(g[:d_x], (0, P - d_x))[None]
    bp = jnp.pad(b[:d_x], (0, P - d_x))[None]
    Wx = _pad2(W[:d_x], P, D)
    full = lambda i: (0, 0)
    out = pl.pallas_call(
        functools.partial(_proj_body, timed, d_x),
        grid=(Np // B,),
        in_specs=[pl.BlockSpec((B, P), lambda i: (i, 0)),
                  pl.BlockSpec((1, 16), full), pl.BlockSpec((1, 16), full),
                  pl.BlockSpec((1, P), full), pl.BlockSpec((1, P), full),
                  pl.BlockSpec((1, 16), full), pl.BlockSpec((1, 16), full),
                  pl.BlockSpec((P, D), full), pl.BlockSpec((16, D), full)],
        out_specs=pl.BlockSpec((B, D), lambda i: (i, 0)),
        out_shape=jax.ShapeDtypeStruct((Np, D), jnp.float32),
    )(xp, wf, bf, gp, bp, gt, btv, Wx, Wt)
    return out[:N]


def _mm_body(x_ref, w_ref, o_ref):
    o_ref[...] = jnp.dot(x_ref[...], w_ref[...],
                         preferred_element_type=jnp.float32)


def _matmul(x, W):
    N, K = x.shape
    M = W.shape[1]
    B = 512
    Np = _ru(max(N, 8), B)
    xp = _pad2(x, Np, K)
    out = pl.pallas_call(
        _mm_body,
        grid=(Np // B,),
        in_specs=[pl.BlockSpec((B, K), lambda i: (i, 0)),
                  pl.BlockSpec((K, M), lambda i: (0, 0))],
        out_specs=pl.BlockSpec((B, M), lambda i: (i, 0)),
        out_shape=jax.ShapeDtypeStruct((Np, M), jnp.float32),
    )(xp, W)
    return out[:N]


def _edge_body(q_ref, kv_ref, r_ref, sel_ref, selT_ref, ev_ref, el_ref):
    q = q_ref[...]
    k = kv_ref[:, :D] + r_ref[...]
    v = kv_ref[:, D:]
    lg = jnp.dot(q * k, sel_ref[...], preferred_element_type=jnp.float32) * _ISQ
    lg = jnp.where(lg > 0, lg, 0.01 * lg)
    col = lax.broadcasted_iota(jnp.int32, lg.shape, 1)
    el = jnp.where(col < H, jnp.exp(lg), 0.0)
    el_ref[...] = el
    ev_ref[...] = v * jnp.dot(el, selT_ref[...],
                              preferred_element_type=jnp.float32)


def _edge_vals(qe, kve, rflat):
    E = qe.shape[0]
    B = 512
    Ep = _ru(max(E, 8), B)
    qp = _pad2(qe, Ep, D)
    kvp = _pad2(kve, Ep, 2 * D)
    full = lambda i: (0, 0)
    ev, el = pl.pallas_call(
        _edge_body,
        grid=(Ep // B,),
        in_specs=[pl.BlockSpec((B, D), lambda i: (i, 0)),
                  pl.BlockSpec((B, 2 * D), lambda i: (i, 0)),
                  pl.BlockSpec((1, D), full),
                  pl.BlockSpec((D, 8), full), pl.BlockSpec((8, D), full)],
        out_specs=[pl.BlockSpec((B, D), lambda i: (i, 0)),
                   pl.BlockSpec((B, 8), lambda i: (i, 0))],
        out_shape=[jax.ShapeDtypeStruct((Ep, D), jnp.float32),
                   jax.ShapeDtypeStruct((Ep, 8), jnp.float32)],
    )(qp, kvp, rflat, jnp.asarray(_SEL), jnp.asarray(_SELT))
    return ev[:E], el[:E]


def _node_body(n_msgs, h_ref, *refs):
    rest = list(refs)
    P1 = rest[:2 * n_msgs]
    selT_ref, g_ref, b_ref, W1_ref, W2_ref, o_ref = rest[2 * n_msgs:]
    h = h_ref[...]
    msg = jnp.zeros_like(h)
    for i in range(n_msgs):
        Pm = P1[2 * i][...]
        sm = P1[2 * i + 1][...]
        den = jnp.dot(sm, selT_ref[...],
                      preferred_element_type=jnp.float32) + 1e-16
        msg = msg + Pm / den
    hr = h + msg
    one = jnp.ones((1, D), jnp.float32)
    zero = jnp.zeros((1, D), jnp.float32)
    z = _ln_exact(hr, one, zero)
    z = _ln_exact(z, g_ref[...], b_ref[...])
    ff = jnp.dot(_gelu(jnp.dot(z, W1_ref[...],
                               preferred_element_type=jnp.float32)),
                 W2_ref[...], preferred_element_type=jnp.float32)
    o_ref[...] = hr + ff


def _node_update(h, msgs, g, b, W1, W2):
    N = h.shape[0]
    B = 256
    Np = _ru(max(N, 8), B)
    hp = _pad2(h, Np, D)
    args = [hp]
    in_specs = [pl.BlockSpec((B, D), lambda i: (i, 0))]
    for (Pm, sm) in msgs:
        args.append(_pad2(Pm, Np, D))
        in_specs.append(pl.BlockSpec((B, D), lambda i: (i, 0)))
        args.append(_pad2(sm, Np, 8))
        in_specs.append(pl.BlockSpec((B, 8), lambda i: (i, 0)))
    full = lambda i: (0, 0)
    args += [jnp.asarray(_SELT), g[None], b[None], W1, W2]
    in_specs += [pl.BlockSpec((8, D), full), pl.BlockSpec((1, D), full),
                 pl.BlockSpec((1, D), full), pl.BlockSpec((D, 4 * D), full),
                 pl.BlockSpec((4 * D, D), full)]
    out = pl.pallas_call(
        functools.partial(_node_body, len(msgs)),
        grid=(Np // B,),
        in_specs=in_specs,
        out_specs=pl.BlockSpec((B, D), lambda i: (i, 0)),
        out_shape=jax.ShapeDtypeStruct((Np, D), jnp.float32),
    )(*args)
    return out[:N]


def _vote_body(raw_ref, pol_ref, hg_ref, W1_ref, b1_ref, W2_ref, b2_ref,
               o_ref):
    e1 = jnp.maximum(jnp.dot(raw_ref[...], W1_ref[...],
                             preferred_element_type=jnp.float32) + b1_ref[...],
                     0.0)
    pol = jnp.clip(pol_ref[...], 0.0, 1.0)
    ef = (jnp.dot(e1, W2_ref[...], preferred_element_type=jnp.float32)
          + b2_ref[...]) * (pol + 0.01)
    o_ref[...] = ef * hg_ref[...]


def _vote_vals(raw, pol, hg, W1, b1, W2, b2):
    E = raw.shape[0]
    B = 512
    Ep = _ru(E, B)
    full = lambda i: (0, 0)
    out = pl.pallas_call(
        _vote_body,
        grid=(Ep // B,),
        in_specs=[pl.BlockSpec((B, 384), lambda i: (i, 0)),
                  pl.BlockSpec((B, 1), lambda i: (i, 0)),
                  pl.BlockSpec((B, D), lambda i: (i, 0)),
                  pl.BlockSpec((384, D), full), pl.BlockSpec((1, D), full),
                  pl.BlockSpec((D, D), full), pl.BlockSpec((1, D), full)],
        out_specs=pl.BlockSpec((B, D), lambda i: (i, 0)),
        out_shape=jax.ShapeDtypeStruct((Ep, D), jnp.float32),
    )(_pad2(raw, Ep, 384), _pad2(pol, Ep, 1), _pad2(hg, Ep, D),
      W1, b1[None], W2, b2[None])
    return out[:E]


def _final_body(h_ref, g_ref, b_ref, o_ref):
    o_ref[...] = jnp.maximum(_ln_exact(h_ref[...], g_ref[...], b_ref[...]),
                             0.0)


def _final_norm(h, g, b):
    N = h.shape[0]
    B = 256
    Np = _ru(max(N, 8), B)
    full = lambda i: (0, 0)
    out = pl.pallas_call(
        _final_body,
        grid=(Np // B,),
        in_specs=[pl.BlockSpec((B, D), lambda i: (i, 0)),
                  pl.BlockSpec((1, D), full), pl.BlockSpec((1, D), full)],
        out_specs=pl.BlockSpec((B, D), lambda i: (i, 0)),
        out_shape=jax.ShapeDtypeStruct((Np, D), jnp.float32),
    )(_pad2(h, Np, D), g[None], b[None])
    return out[:N]


_CH = 128


def _sc_gather(table, idx):
    E = idx.shape[0]
    W = table.shape[1]
    assert E % (32 * _CH) == 0
    e_per_w = E // 32
    nch = e_per_w // _CH
    mesh = plsc.VectorSubcoreMesh(core_axis_name="c", subcore_axis_name="s")

    @functools.partial(
        pl.kernel, mesh=mesh,
        compiler_params=pltpu.CompilerParams(use_tc_tiling_on_sc=False),
        out_type=jax.ShapeDtypeStruct((E, W), jnp.float32),
        scratch_types=[pltpu.VMEM((_CH,), jnp.int32),
                       pltpu.VMEM((_CH, W), jnp.float32),
                       pltpu.SemaphoreType.DMA],
    )
    def gk(table_hbm, idx_hbm, out_hbm, idx_v, rows_v, sem):
        wid = lax.axis_index("s") * 2 + lax.axis_index("c")
        base = wid * e_per_w

        def body(i, carry):
            off = base + i * _CH
            pltpu.sync_copy(idx_hbm.at[pl.ds(off, _CH)], idx_v)
            pltpu.async_copy(table_hbm.at[idx_v], rows_v, sem).wait()
            pltpu.sync_copy(rows_v, out_hbm.at[pl.ds(off, _CH)])
            return carry

        lax.fori_loop(0, nch, body, 0)

    return gk(table, idx)


def _sc_scatter_mat(ev, dst, n):
    E = ev.shape[0]
    assert E % (16 * _CH) == 0
    e_per_t = E // 16
    nch = e_per_t // _CH
    nacc = _ru(n + 1, 16 * _CH)
    cw = 96 if nacc * 96 * 4 <= 7 * 1024 * 1024 else 32
    njc = (D // cw) // 2
    rpt = nacc // 16
    nrb = rpt // _CH
    mesh = plsc.VectorSubcoreMesh(core_axis_name="c", subcore_axis_name="s")

    @functools.partial(
        pl.kernel, mesh=mesh,
        compiler_params=pltpu.CompilerParams(use_tc_tiling_on_sc=False),
        out_type=jax.ShapeDtypeStruct((nacc, D), jnp.float32),
        scratch_types=[pltpu.VMEM((_CH,), jnp.int32),
                       pltpu.VMEM((_CH, cw), jnp.float32),
                       pltpu.VMEM((_CH, cw), jnp.float32),
                       pltpu.VMEM_SHARED((nacc, cw), jnp.float32),
                       pltpu.SemaphoreType.DMA],
    )
    def sk(zeros_hbm, ev_hbm, dst_hbm, out_hbm, idx_v, vals_v, zero_v,
           acc_sh, sem):
        cid = lax.axis_index("c")
        sid = lax.axis_index("s")
        pltpu.sync_copy(zeros_hbm, zero_v)
        ebase = sid * e_per_t
        rbase = sid * rpt
        for j in range(njc):
            coff = (2 * j + cid) * cw

            def zbody(r, carry):
                pltpu.sync_copy(zero_v, acc_sh.at[pl.ds(rbase + r * _CH, _CH)])
                return carry

            lax.fori_loop(0, nrb, zbody, 0)
            plsc.subcore_barrier()

            def ebody(i, carry):
                off = ebase + i * _CH
                pltpu.sync_copy(dst_hbm.at[pl.ds(off, _CH)], idx_v)
                pltpu.sync_copy(
                    ev_hbm.at[pl.ds(off, _CH), pl.ds(coff, cw)], vals_v)
                pltpu.sync_copy(vals_v, acc_sh.at[idx_v], add=True)
                return carry

            lax.fori_loop(0, nch, ebody, 0)
            plsc.subcore_barrier()

            def obody(r, carry):
                r0 = rbase + r * _CH
                pltpu.sync_copy(acc_sh.at[pl.ds(r0, _CH)], vals_v)
                pltpu.sync_copy(vals_v,
                                out_hbm.at[pl.ds(r0, _CH), pl.ds(coff, cw)])
                return carry

            lax.fori_loop(0, nrb, obody, 0)
            plsc.subcore_barrier()

    out = sk(jnp.zeros((_CH, cw), jnp.float32), ev, dst)
    return out[:n]


def _sc_scatter_vec(el, dst, n):
    E = el.shape[0]
    W = el.shape[1]
    assert E % (16 * _CH) == 0
    e_per_t = E // 16
    nch = e_per_t // _CH
    nacc = _ru(n + 1, 16 * _CH)
    rpt = nacc // 16
    nrb = rpt // _CH
    mesh = plsc.VectorSubcoreMesh(core_axis_name="c", subcore_axis_name="s")

    @functools.partial(
        pl.kernel, mesh=mesh,
        compiler_params=pltpu.CompilerParams(use_tc_tiling_on_sc=False),
        out_type=jax.ShapeDtypeStruct((nacc, W), jnp.float32),
        scratch_types=[pltpu.VMEM((_CH,), jnp.int32),
                       pltpu.VMEM((_CH, W), jnp.float32),
                       pltpu.VMEM((_CH, W), jnp.float32),
                       pltpu.VMEM_SHARED((nacc, W), jnp.float32),
                       pltpu.SemaphoreType.DMA],
    )
    def sk(zeros_hbm, el_hbm, dst_hbm, out_hbm, idx_v, vals_v, zero_v,
           acc_sh, sem):
        cid = lax.axis_index("c")
        sid = lax.axis_index("s")

        @pl.when(cid == 0)
        def _():
            pltpu.sync_copy(zeros_hbm, zero_v)
            ebase = sid * e_per_t
            rbase = sid * rpt

            def zbody(r, carry):
                pltpu.sync_copy(zero_v, acc_sh.at[pl.ds(rbase + r * _CH, _CH)])
                return carry

            lax.fori_loop(0, nrb, zbody, 0)
            plsc.subcore_barrier()

            def ebody(i, carry):
                off = ebase + i * _CH
                pltpu.sync_copy(dst_hbm.at[pl.ds(off, _CH)], idx_v)
                pltpu.sync_copy(el_hbm.at[pl.ds(off, _CH)], vals_v)
                pltpu.sync_copy(vals_v, acc_sh.at[idx_v], add=True)
                return carry

            lax.fori_loop(0, nch, ebody, 0)
            plsc.subcore_barrier()

            def obody(r, carry):
                r0 = rbase + r * _CH
                pltpu.sync_copy(acc_sh.at[pl.ds(r0, _CH)], vals_v)
                pltpu.sync_copy(vals_v, out_hbm.at[pl.ds(r0, _CH)])
                return carry

            lax.fori_loop(0, nrb, obody, 0)

    out = sk(jnp.zeros((_CH, W), jnp.float32), el, dst)
    return out[:n]


def kernel(x_bill, x_bill_version, x_legislator_term, x_legislator,
           x_committee, x_party, x_topic, ts_bill, ts_bill_version,
           ts_legislator_term, ei_has_version, ei_voted_on, ei_serves,
           ei_about, ea_voted_on, t2v_w0, t2v_w, t2v_b,
           prj_ln_g_bill, prj_ln_b_bill, prj_W_bill,
           prj_ln_g_bill_version, prj_ln_b_bill_version, prj_W_bill_version,
           prj_ln_g_legislator_term, prj_ln_b_legislator_term,
           prj_W_legislator_term, prj_ln_g_legislator, prj_ln_b_legislator,
           prj_W_legislator, prj_ln_g_committee, prj_ln_b_committee,
           prj_W_committee, prj_ln_g_party, prj_ln_b_party, prj_W_party,
           prj_ln_g_topic, prj_ln_b_topic, prj_W_topic,
           Q_0, K_0, V_0, rel_0_has_version, rel_0_voted_on, rel_0_serves,
           rel_0_about, ffn_ln_g_0, ffn_ln_b_0, ffn_W1_0, ffn_W2_0,
           Q_1, K_1, V_1, rel_1_has_version, rel_1_voted_on, rel_1_serves,
           rel_1_about, ffn_ln_g_1, ffn_ln_b_1, ffn_W1_1, ffn_W2_1,
           Q_2, K_2, V_2, rel_2_has_version, rel_2_voted_on, rel_2_serves,
           rel_2_about, ffn_ln_g_2, ffn_ln_b_2, ffn_W1_2, ffn_W2_2,
           vote_W1, vote_b1, vote_W2, vote_b2,
           norm_g_bill, norm_b_bill, norm_g_bill_version, norm_b_bill_version,
           norm_g_legislator_term, norm_b_legislator_term, norm_g_legislator,
           norm_b_legislator, norm_g_committee, norm_b_committee,
           norm_g_party, norm_b_party, norm_g_topic, norm_b_topic):
    p = dict(locals())
    h = {}
    for nt in _NODE_TYPES:
        timed = nt in _TIME_TYPES
        h[nt] = _project(p["x_" + nt],
                         p.get("ts_" + nt) if timed else None,
                         p["prj_ln_g_" + nt], p["prj_ln_b_" + nt],
                         p["prj_W_" + nt], t2v_w0, t2v_w, t2v_b, timed)
    nnodes = {nt: h[nt].shape[0] for nt in _NODE_TYPES}
    ei = {r: p["ei_" + r].astype(jnp.int32) for (_, r, _) in _EDGE_TYPES}
    src_types = {s for (s, _, _) in _EDGE_TYPES}
    dst_types = {t for (_, _, t) in _EDGE_TYPES}
    for l in range(3):
        Qw, Kw, Vw = p["Q_%d" % l], p["K_%d" % l], p["V_%d" % l]
        KVw = jnp.concatenate([Kw, Vw], axis=1)
        Qh = {t: _matmul(h[t], Qw) for t in dst_types}
        KVh = {s: _matmul(h[s], KVw) for s in src_types}
        msgs = {t: [] for t in _NODE_TYPES}
        for (sname, rname, tname) in _EDGE_TYPES:
            e = ei[rname]
            E = e.shape[1]
            EP = _ru(E, 32 * _CH)
            nt = nnodes[tname]
            src0 = jnp.pad(e[0], (0, EP - E))
            dst0 = jnp.pad(e[1], (0, EP - E))
            dstn = jnp.pad(e[1], (0, EP - E), constant_values=nt)
            qe = _sc_gather(Qh[tname], dst0)
            kve = _sc_gather(KVh[sname], src0)
            rflat = p["rel_%d_%s" % (l, rname)].reshape(1, D)
            ev, el = _edge_vals(qe, kve, rflat)
            Pn = _sc_scatter_mat(ev, dstn, nt)
            sn = _sc_scatter_vec(el, dstn, nt)
            msgs[tname].append((Pn, sn))
        h = {nt: _node_update(h[nt], msgs[nt], p["ffn_ln_g_%d" % l],
                              p["ffn_ln_b_%d" % l], p["ffn_W1_%d" % l],
                              p["ffn_W2_%d" % l])
             for nt in _NODE_TYPES}
    ev_ei = ei["voted_on"]
    E = ev_ei.shape[1]
    EP = _ru(E, 32 * _CH)
    nbv = nnodes["bill_version"]
    src0 = jnp.pad(ev_ei[0], (0, EP - E))
    dstn = jnp.pad(ev_ei[1], (0, EP - E), constant_values=nbv)
    hg = _sc_gather(h["legislator_term"], src0)
    m = _vote_vals(_pad2(ea_voted_on[:, 1:], EP, 384),
                   _pad2(ea_voted_on[:, :1], EP, 1), hg,
                   vote_W1, vote_b1, vote_W2, vote_b2)
    vmsg = _sc_scatter_mat(m, dstn, nbv)
    h["bill_version"] = h["bill_version"] + vmsg
    return tuple(_final_norm(h[nt], p["norm_g_" + nt], p["norm_b_" + nt])
                 for nt in _NODE_TYPES)


# speedup vs baseline: 2.1620x; 1.1790x over previous
import functools

import numpy as np
import jax
import jax.numpy as jnp
from jax import lax
from jax.experimental import pallas as pl
from jax.experimental.pallas import tpu as pltpu
from jax.experimental.pallas import tpu_sc as plsc

D = 192
H = 4
DK = 48
_ISQ = float(1.0 / np.sqrt(DK))
_NODE_TYPES = ["bill", "bill_version", "legislator_term", "legislator",
               "committee", "party", "topic"]
_IN_DIMS = {"bill": 389, "bill_version": 769, "legislator_term": 385,
            "legislator": 2, "committee": 385, "party": 384, "topic": 384}
_TIME_TYPES = ("bill", "bill_version", "legislator_term")
_EDGE_TYPES = [("bill", "has_version", "bill_version"),
               ("legislator_term", "voted_on", "bill_version"),
               ("legislator", "serves", "legislator_term"),
               ("bill", "about", "topic")]

_SEL = np.zeros((D, 8), np.float32)
for _d in range(D):
    _SEL[_d, _d // DK] = 1.0
_SELT = _SEL.T.copy()


def _ru(x, m):
    return (x + m - 1) // m * m


def _pad2(a, r, c):
    return jnp.pad(a, ((0, r - a.shape[0]), (0, c - a.shape[1])))


def _ln_exact(x, g, b):
    mu = jnp.mean(x, axis=1, keepdims=True)
    xc = x - mu
    var = jnp.mean(xc * xc, axis=1, keepdims=True)
    return xc * lax.rsqrt(var + 1e-5) * g + b


def _gelu(x):
    return 0.5 * x * (1.0 + lax.erf(x * np.float32(1.0 / np.sqrt(2.0))))


def _proj_body(timed, d_x, x_ref, wf_ref, bf_ref, g_ref, b_ref, gt_ref,
               bt_ref, Wx_ref, Wt_ref, o_ref):
    xb = x_ref[...]
    colx = lax.broadcasted_iota(jnp.int32, xb.shape, 1)
    xm = jnp.where(colx < d_x, xb, 0.0)
    s1 = jnp.sum(xm, axis=1, keepdims=True)
    s2 = jnp.sum(xm * xm, axis=1, keepdims=True)
    if timed:
        t = xb[:, d_x:d_x + 1]
        tv = t * wf_ref[...] + bf_ref[...]
        c16 = lax.broadcasted_iota(jnp.int32, tv.shape, 1)
        v = jnp.where(c16 == 0, tv, jnp.sin(tv))
        v = jnp.where(c16 < 12, v, 0.0)
        s1 = s1 + jnp.sum(v, axis=1, keepdims=True)
        s2 = s2 + jnp.sum(v * v, axis=1, keepdims=True)
        n = d_x + 12
    else:
        n = d_x
    mu = s1 / n
    var = jnp.maximum(s2 / n - mu * mu, 0.0)
    r = lax.rsqrt(var + 1e-5)
    zx = ((xm - mu) * r) * g_ref[...] + b_ref[...]
    zx = jnp.where(colx < d_x, zx, 0.0)
    acc = jnp.dot(zx, Wx_ref[...], preferred_element_type=jnp.float32)
    if timed:
        zv = ((v - mu) * r) * gt_ref[...] + bt_ref[...]
        acc = acc + jnp.dot(zv, Wt_ref[...], preferred_element_type=jnp.float32)
    o_ref[...] = _gelu(acc)


def _project(x, ts, g, b, W, w0, wt, bt, timed):
    N, d_x = x.shape
    B = 256
    P = _ru(d_x + 1, 128)
    Np = _ru(max(N, 8), B)
    if timed:
        xc = jnp.concatenate([x, ts[:, None]], axis=1)
        wf = jnp.concatenate([w0[None], wt, jnp.zeros((4,), jnp.float32)])[None]
        bf = jnp.concatenate([bt, jnp.zeros((4,), jnp.float32)])[None]
        gt = jnp.pad(g[d_x:d_x + 12], (0, 4))[None]
        btv = jnp.pad(b[d_x:d_x + 12], (0, 4))[None]
        Wt = jnp.pad(W[d_x:d_x + 12], ((0, 4), (0, 0)))
    else:
        xc = x
        wf = jnp.zeros((1, 16), jnp.float32)
        bf = jnp.zeros((1, 16), jnp.float32)
        gt = jnp.zeros((1, 16), jnp.float32)
        btv = jnp.zeros((1, 16), jnp.float32)
        Wt = jnp.zeros((16, D), jnp.float32)
    xp = _pad2(xc, Np, P)
    gp = jnp.pad(g[:d_x], (0, P - d_x))[None]
    bp = jnp.pad(b[:d_x], (0, P - d_x))[None]
    Wx = _pad2(W[:d_x], P, D)
    full = lambda i: (0, 0)
    out = pl.pallas_call(
        functools.partial(_proj_body, timed, d_x),
        grid=(Np // B,),
        in_specs=[pl.BlockSpec((B, P), lambda i: (i, 0)),
                  pl.BlockSpec((1, 16), full), pl.BlockSpec((1, 16), full),
                  pl.BlockSpec((1, P), full), pl.BlockSpec((1, P), full),
                  pl.BlockSpec((1, 16), full), pl.BlockSpec((1, 16), full),
                  pl.BlockSpec((P, D), full), pl.BlockSpec((16, D), full)],
        out_specs=pl.BlockSpec((B, D), lambda i: (i, 0)),
        out_shape=jax.ShapeDtypeStruct((Np, D), jnp.float32),
    )(xp, wf, bf, gp, bp, gt, btv, Wx, Wt)
    return out[:N]


def _mm_body(x_ref, w_ref, o_ref):
    o_ref[...] = jnp.dot(x_ref[...], w_ref[...],
                         preferred_element_type=jnp.float32)


def _matmul(x, W):
    N, K = x.shape
    M = W.shape[1]
    B = 512
    Np = _ru(max(N, 8), B)
    xp = _pad2(x, Np, K)
    out = pl.pallas_call(
        _mm_body,
        grid=(Np // B,),
        in_specs=[pl.BlockSpec((B, K), lambda i: (i, 0)),
                  pl.BlockSpec((K, M), lambda i: (0, 0))],
        out_specs=pl.BlockSpec((B, M), lambda i: (i, 0)),
        out_shape=jax.ShapeDtypeStruct((Np, M), jnp.float32),
    )(xp, W)
    return out[:N]


def _edge_body(q_ref, kv_ref, r_ref, sel_ref, selT_ref, ev_ref, el_ref):
    q = q_ref[:, :D]
    k = kv_ref[:, :D] + r_ref[...]
    v = kv_ref[:, D:]
    lg = jnp.dot(q * k, sel_ref[...], preferred_element_type=jnp.float32) * _ISQ
    lg = jnp.where(lg > 0, lg, 0.01 * lg)
    col = lax.broadcasted_iota(jnp.int32, lg.shape, 1)
    el = jnp.where(col < H, jnp.exp(lg), 0.0)
    el_ref[...] = el
    ev_ref[...] = v * jnp.dot(el, selT_ref[...],
                              preferred_element_type=jnp.float32)


def _edge_vals(qe, kve, rflat):
    E = qe.shape[0]
    B = 512
    Ep = _ru(max(E, 8), B)
    qp = _pad2(qe, Ep, 256)
    kvp = _pad2(kve, Ep, 2 * D)
    full = lambda i: (0, 0)
    ev, el = pl.pallas_call(
        _edge_body,
        grid=(Ep // B,),
        in_specs=[pl.BlockSpec((B, 256), lambda i: (i, 0)),
                  pl.BlockSpec((B, 2 * D), lambda i: (i, 0)),
                  pl.BlockSpec((1, D), full),
                  pl.BlockSpec((D, 8), full), pl.BlockSpec((8, D), full)],
        out_specs=[pl.BlockSpec((B, D), lambda i: (i, 0)),
                   pl.BlockSpec((B, 8), lambda i: (i, 0))],
        out_shape=[jax.ShapeDtypeStruct((Ep, D), jnp.float32),
                   jax.ShapeDtypeStruct((Ep, 8), jnp.float32)],
    )(qp, kvp, rflat, jnp.asarray(_SEL), jnp.asarray(_SELT))
    return ev[:E], el[:E]


def _node_body(n_msgs, h_ref, *refs):
    rest = list(refs)
    P1 = rest[:2 * n_msgs]
    selT_ref, g_ref, b_ref, W1_ref, W2_ref, o_ref = rest[2 * n_msgs:]
    h = h_ref[...]
    msg = jnp.zeros_like(h)
    for i in range(n_msgs):
        Pm = P1[2 * i][...]
        sm = P1[2 * i + 1][...]
        den = jnp.dot(sm, selT_ref[...],
                      preferred_element_type=jnp.float32) + 1e-16
        msg = msg + Pm / den
    hr = h + msg
    one = jnp.ones((1, D), jnp.float32)
    zero = jnp.zeros((1, D), jnp.float32)
    z = _ln_exact(hr, one, zero)
    z = _ln_exact(z, g_ref[...], b_ref[...])
    ff = jnp.dot(_gelu(jnp.dot(z, W1_ref[...],
                               preferred_element_type=jnp.float32)),
                 W2_ref[...], preferred_element_type=jnp.float32)
    o_ref[...] = hr + ff


def _node_update(h, msgs, g, b, W1, W2):
    N = h.shape[0]
    B = 256
    Np = _ru(max(N, 8), B)
    hp = _pad2(h, Np, D)
    args = [hp]
    in_specs = [pl.BlockSpec((B, D), lambda i: (i, 0))]
    for (Pm, sm) in msgs:
        args.append(_pad2(Pm, Np, D))
        in_specs.append(pl.BlockSpec((B, D), lambda i: (i, 0)))
        args.append(_pad2(sm, Np, 8))
        in_specs.append(pl.BlockSpec((B, 8), lambda i: (i, 0)))
    full = lambda i: (0, 0)
    args += [jnp.asarray(_SELT), g[None], b[None], W1, W2]
    in_specs += [pl.BlockSpec((8, D), full), pl.BlockSpec((1, D), full),
                 pl.BlockSpec((1, D), full), pl.BlockSpec((D, 4 * D), full),
                 pl.BlockSpec((4 * D, D), full)]
    out = pl.pallas_call(
        functools.partial(_node_body, len(msgs)),
        grid=(Np // B,),
        in_specs=in_specs,
        out_specs=pl.BlockSpec((B, D), lambda i: (i, 0)),
        out_shape=jax.ShapeDtypeStruct((Np, D), jnp.float32),
    )(*args)
    return out[:N]


def _vote_body(raw_ref, pol_ref, hg_ref, W1_ref, b1_ref, W2_ref, b2_ref,
               o_ref):
    e1 = jnp.maximum(jnp.dot(raw_ref[...], W1_ref[...],
                             preferred_element_type=jnp.float32) + b1_ref[...],
                     0.0)
    pol = jnp.clip(pol_ref[...], 0.0, 1.0)
    ef = (jnp.dot(e1, W2_ref[...], preferred_element_type=jnp.float32)
          + b2_ref[...]) * (pol + 0.01)
    o_ref[...] = ef * hg_ref[:, :D]


def _vote_vals(raw, pol, hg, W1, b1, W2, b2):
    E = raw.shape[0]
    B = 512
    Ep = _ru(E, B)
    full = lambda i: (0, 0)
    out = pl.pallas_call(
        _vote_body,
        grid=(Ep // B,),
        in_specs=[pl.BlockSpec((B, 384), lambda i: (i, 0)),
                  pl.BlockSpec((B, 1), lambda i: (i, 0)),
                  pl.BlockSpec((B, 256), lambda i: (i, 0)),
                  pl.BlockSpec((384, D), full), pl.BlockSpec((1, D), full),
                  pl.BlockSpec((D, D), full), pl.BlockSpec((1, D), full)],
        out_specs=pl.BlockSpec((B, D), lambda i: (i, 0)),
        out_shape=jax.ShapeDtypeStruct((Ep, D), jnp.float32),
    )(_pad2(raw, Ep, 384), _pad2(pol, Ep, 1), _pad2(hg, Ep, 256),
      W1, b1[None], W2, b2[None])
    return out[:E]


def _final_body(h_ref, g_ref, b_ref, o_ref):
    o_ref[...] = jnp.maximum(_ln_exact(h_ref[...], g_ref[...], b_ref[...]),
                             0.0)


def _final_norm(h, g, b):
    N = h.shape[0]
    B = 256
    Np = _ru(max(N, 8), B)
    full = lambda i: (0, 0)
    out = pl.pallas_call(
        _final_body,
        grid=(Np // B,),
        in_specs=[pl.BlockSpec((B, D), lambda i: (i, 0)),
                  pl.BlockSpec((1, D), full), pl.BlockSpec((1, D), full)],
        out_specs=pl.BlockSpec((B, D), lambda i: (i, 0)),
        out_shape=jax.ShapeDtypeStruct((Np, D), jnp.float32),
    )(_pad2(h, Np, D), g[None], b[None])
    return out[:N]


_CH = 128


def _sc_gather(table, idx):
    E = idx.shape[0]
    W = table.shape[1]
    assert E % (32 * _CH) == 0 and W % 128 == 0
    e_per_w = E // 32
    nch = e_per_w // _CH
    mesh = plsc.VectorSubcoreMesh(core_axis_name="c", subcore_axis_name="s")

    @functools.partial(
        pl.kernel, mesh=mesh,
        out_type=jax.ShapeDtypeStruct((E, W), jnp.float32),
        scratch_types=[pltpu.VMEM((_CH,), jnp.int32),
                       pltpu.VMEM((_CH, W), jnp.float32),
                       pltpu.SemaphoreType.DMA],
    )
    def gk(table_hbm, idx_hbm, out_hbm, idx_v, rows_v, sem):
        wid = lax.axis_index("s") * 2 + lax.axis_index("c")
        base = wid * e_per_w

        def body(i, carry):
            off = base + i * _CH
            pltpu.sync_copy(idx_hbm.at[pl.ds(off, _CH)], idx_v)
            pltpu.async_copy(table_hbm.at[idx_v], rows_v, sem).wait()
            pltpu.sync_copy(rows_v, out_hbm.at[pl.ds(off, _CH)])
            return carry

        lax.fori_loop(0, nch, body, 0)

    return gk(table, idx)


def _sc_scatter_mat(ev, dst, n):
    E = ev.shape[0]
    assert E % (16 * _CH) == 0
    e_per_t = E // 16
    nch = e_per_t // _CH
    nacc = _ru(n + 1, 16 * _CH)
    cw = 96 if nacc * 96 * 4 <= 7 * 1024 * 1024 else 32
    njc = (D // cw) // 2
    rpt = nacc // 16
    nrb = rpt // _CH
    mesh = plsc.VectorSubcoreMesh(core_axis_name="c", subcore_axis_name="s")

    @functools.partial(
        pl.kernel, mesh=mesh,
        compiler_params=pltpu.CompilerParams(use_tc_tiling_on_sc=False),
        out_type=jax.ShapeDtypeStruct((nacc, D), jnp.float32),
        scratch_types=[pltpu.VMEM((_CH,), jnp.int32),
                       pltpu.VMEM((_CH, cw), jnp.float32),
                       pltpu.VMEM((_CH, cw), jnp.float32),
                       pltpu.VMEM_SHARED((nacc, cw), jnp.float32),
                       pltpu.SemaphoreType.DMA],
    )
    def sk(zeros_hbm, ev_hbm, dst_hbm, out_hbm, idx_v, vals_v, zero_v,
           acc_sh, sem):
        cid = lax.axis_index("c")
        sid = lax.axis_index("s")
        pltpu.sync_copy(zeros_hbm, zero_v)
        ebase = sid * e_per_t
        rbase = sid * rpt
        for j in range(njc):
            coff = (2 * j + cid) * cw

            def zbody(r, carry):
                pltpu.sync_copy(zero_v, acc_sh.at[pl.ds(rbase + r * _CH, _CH)])
                return carry

            lax.fori_loop(0, nrb, zbody, 0)
            plsc.subcore_barrier()

            def ebody(i, carry):
                off = ebase + i * _CH
                pltpu.sync_copy(dst_hbm.at[pl.ds(off, _CH)], idx_v)
                pltpu.sync_copy(
                    ev_hbm.at[pl.ds(off, _CH), pl.ds(coff, cw)], vals_v)
                pltpu.sync_copy(vals_v, acc_sh.at[idx_v], add=True)
                return carry

            lax.fori_loop(0, nch, ebody, 0)
            plsc.subcore_barrier()

            def obody(r, carry):
                r0 = rbase + r * _CH
                pltpu.sync_copy(acc_sh.at[pl.ds(r0, _CH)], vals_v)
                pltpu.sync_copy(vals_v,
                                out_hbm.at[pl.ds(r0, _CH), pl.ds(coff, cw)])
                return carry

            lax.fori_loop(0, nrb, obody, 0)
            plsc.subcore_barrier()

    out = sk(jnp.zeros((_CH, cw), jnp.float32), ev, dst)
    return out[:n]


def _sc_scatter_vec(el, dst, n):
    E = el.shape[0]
    W = el.shape[1]
    assert E % (16 * _CH) == 0
    e_per_t = E // 16
    nch = e_per_t // _CH
    nacc = _ru(n + 1, 16 * _CH)
    rpt = nacc // 16
    nrb = rpt // _CH
    mesh = plsc.VectorSubcoreMesh(core_axis_name="c", subcore_axis_name="s")

    @functools.partial(
        pl.kernel, mesh=mesh,
        compiler_params=pltpu.CompilerParams(use_tc_tiling_on_sc=False),
        out_type=jax.ShapeDtypeStruct((nacc, W), jnp.float32),
        scratch_types=[pltpu.VMEM((_CH,), jnp.int32),
                       pltpu.VMEM((_CH, W), jnp.float32),
                       pltpu.VMEM((_CH, W), jnp.float32),
                       pltpu.VMEM_SHARED((nacc, W), jnp.float32),
                       pltpu.SemaphoreType.DMA],
    )
    def sk(zeros_hbm, el_hbm, dst_hbm, out_hbm, idx_v, vals_v, zero_v,
           acc_sh, sem):
        cid = lax.axis_index("c")
        sid = lax.axis_index("s")

        @pl.when(cid == 0)
        def _():
            pltpu.sync_copy(zeros_hbm, zero_v)
            ebase = sid * e_per_t
            rbase = sid * rpt

            def zbody(r, carry):
                pltpu.sync_copy(zero_v, acc_sh.at[pl.ds(rbase + r * _CH, _CH)])
                return carry

            lax.fori_loop(0, nrb, zbody, 0)
            plsc.subcore_barrier()

            def ebody(i, carry):
                off = ebase + i * _CH
                pltpu.sync_copy(dst_hbm.at[pl.ds(off, _CH)], idx_v)
                pltpu.sync_copy(el_hbm.at[pl.ds(off, _CH)], vals_v)
                pltpu.sync_copy(vals_v, acc_sh.at[idx_v], add=True)
                return carry

            lax.fori_loop(0, nch, ebody, 0)
            plsc.subcore_barrier()

            def obody(r, carry):
                r0 = rbase + r * _CH
                pltpu.sync_copy(acc_sh.at[pl.ds(r0, _CH)], vals_v)
                pltpu.sync_copy(vals_v, out_hbm.at[pl.ds(r0, _CH)])
                return carry

            lax.fori_loop(0, nrb, obody, 0)

    out = sk(jnp.zeros((_CH, W), jnp.float32), el, dst)
    return out[:n]


def kernel(x_bill, x_bill_version, x_legislator_term, x_legislator,
           x_committee, x_party, x_topic, ts_bill, ts_bill_version,
           ts_legislator_term, ei_has_version, ei_voted_on, ei_serves,
           ei_about, ea_voted_on, t2v_w0, t2v_w, t2v_b,
           prj_ln_g_bill, prj_ln_b_bill, prj_W_bill,
           prj_ln_g_bill_version, prj_ln_b_bill_version, prj_W_bill_version,
           prj_ln_g_legislator_term, prj_ln_b_legislator_term,
           prj_W_legislator_term, prj_ln_g_legislator, prj_ln_b_legislator,
           prj_W_legislator, prj_ln_g_committee, prj_ln_b_committee,
           prj_W_committee, prj_ln_g_party, prj_ln_b_party, prj_W_party,
           prj_ln_g_topic, prj_ln_b_topic, prj_W_topic,
           Q_0, K_0, V_0, rel_0_has_version, rel_0_voted_on, rel_0_serves,
           rel_0_about, ffn_ln_g_0, ffn_ln_b_0, ffn_W1_0, ffn_W2_0,
           Q_1, K_1, V_1, rel_1_has_version, rel_1_voted_on, rel_1_serves,
           rel_1_about, ffn_ln_g_1, ffn_ln_b_1, ffn_W1_1, ffn_W2_1,
           Q_2, K_2, V_2, rel_2_has_version, rel_2_voted_on, rel_2_serves,
           rel_2_about, ffn_ln_g_2, ffn_ln_b_2, ffn_W1_2, ffn_W2_2,
           vote_W1, vote_b1, vote_W2, vote_b2,
           norm_g_bill, norm_b_bill, norm_g_bill_version, norm_b_bill_version,
           norm_g_legislator_term, norm_b_legislator_term, norm_g_legislator,
           norm_b_legislator, norm_g_committee, norm_b_committee,
           norm_g_party, norm_b_party, norm_g_topic, norm_b_topic):
    p = dict(locals())
    h = {}
    for nt in _NODE_TYPES:
        timed = nt in _TIME_TYPES
        h[nt] = _project(p["x_" + nt],
                         p.get("ts_" + nt) if timed else None,
                         p["prj_ln_g_" + nt], p["prj_ln_b_" + nt],
                         p["prj_W_" + nt], t2v_w0, t2v_w, t2v_b, timed)
    nnodes = {nt: h[nt].shape[0] for nt in _NODE_TYPES}
    ei = {r: p["ei_" + r].astype(jnp.int32) for (_, r, _) in _EDGE_TYPES}
    src_types = {s for (s, _, _) in _EDGE_TYPES}
    dst_types = {t for (_, _, t) in _EDGE_TYPES}
    for l in range(3):
        Qw, Kw, Vw = p["Q_%d" % l], p["K_%d" % l], p["V_%d" % l]
        KVw = jnp.concatenate([Kw, Vw], axis=1)
        Qwp = _pad2(Qw, D, 256)
        Qh = {t: _matmul(h[t], Qwp) for t in dst_types}
        KVh = {s: _matmul(h[s], KVw) for s in src_types}
        msgs = {t: [] for t in _NODE_TYPES}
        for (sname, rname, tname) in _EDGE_TYPES:
            e = ei[rname]
            E = e.shape[1]
            EP = _ru(E, 32 * _CH)
            nt = nnodes[tname]
            src0 = jnp.pad(e[0], (0, EP - E))
            dst0 = jnp.pad(e[1], (0, EP - E))
            dstn = jnp.pad(e[1], (0, EP - E), constant_values=nt)
            qe = _sc_gather(Qh[tname], dst0)
            kve = _sc_gather(KVh[sname], src0)
            rflat = p["rel_%d_%s" % (l, rname)].reshape(1, D)
            ev, el = _edge_vals(qe, kve, rflat)
            Pn = _sc_scatter_mat(ev, dstn, nt)
            sn = _sc_scatter_vec(el, dstn, nt)
            msgs[tname].append((Pn, sn))
        h = {nt: _node_update(h[nt], msgs[nt], p["ffn_ln_g_%d" % l],
                              p["ffn_ln_b_%d" % l], p["ffn_W1_%d" % l],
                              p["ffn_W2_%d" % l])
             for nt in _NODE_TYPES}
    ev_ei = ei["voted_on"]
    E = ev_ei.shape[1]
    EP = _ru(E, 32 * _CH)
    nbv = nnodes["bill_version"]
    src0 = jnp.pad(ev_ei[0], (0, EP - E))
    dstn = jnp.pad(ev_ei[1], (0, EP - E), constant_values=nbv)
    hg = _sc_gather(_pad2(h["legislator_term"],
                          h["legislator_term"].shape[0], 256), src0)
    m = _vote_vals(_pad2(ea_voted_on[:, 1:], EP, 384),
                   _pad2(ea_voted_on[:, :1], EP, 1), hg,
                   vote_W1, vote_b1, vote_W2, vote_b2)
    vmsg = _sc_scatter_mat(m, dstn, nbv)
    h["bill_version"] = h["bill_version"] + vmsg
    return tuple(_final_norm(h[nt], p["norm_g_" + nt], p["norm_b_" + nt])
                 for nt in _NODE_TYPES)
